# baseline (speedup 1.0000x reference)
import functools

import jax
import jax.numpy as jnp
from jax import lax
from jax.experimental import pallas as pl
from jax.experimental.pallas import tpu as pltpu
from jax.experimental.pallas import tpu_sc as plsc

VOCAB = 100000
EMBED = 128
BATCH = 4096
SEQ = 200

NC = 2
NS = 16
NW = NC * NS

TOTAL = BATCH * SEQ
B_PER_W = TOTAL // NW
CH = 128
N_CH = B_PER_W // CH
NBUF = 5


def _sc_gather(x_resh, table):
    mesh = plsc.VectorSubcoreMesh(core_axis_name="c", subcore_axis_name="s")

    @functools.partial(
        pl.kernel,
        mesh=mesh,
        out_type=jax.ShapeDtypeStruct((NW, N_CH, CH, EMBED), jnp.float32),
        scratch_types=[
            pltpu.VMEM((N_CH, CH), jnp.int32),
            pltpu.VMEM((NBUF, CH, EMBED), jnp.float32),
            pltpu.SemaphoreType.DMA((NBUF,)),
            pltpu.SemaphoreType.DMA((NBUF,)),
        ],
    )
    def k(idx_hbm, table_hbm, out_hbm, idx_v, rows_v, sem_g, sem_o):
        wid = lax.axis_index("s") * NC + lax.axis_index("c")
        pltpu.sync_copy(idx_hbm.at[wid], idx_v)

        def fire_gather(chunk, b):
            pltpu.async_copy(table_hbm.at[idx_v.at[chunk]], rows_v.at[b],
                             sem_g.at[b])

        def fire_out(chunk, b):
            pltpu.async_copy(rows_v.at[b], out_hbm.at[wid, chunk],
                             sem_o.at[b])

        def drain(sem, b):
            pltpu.make_async_copy(out_hbm.at[wid, 0], rows_v.at[b],
                                  sem.at[b]).wait()

        for b in range(NBUF):
            fire_gather(b, b)

        @pl.loop(0, N_CH - NBUF, step=NBUF)
        def _ring(t):
            for b in range(NBUF):
                drain(sem_g, b)
                fire_out(t + b, b)
            for b in range(NBUF):
                drain(sem_o, b)
                fire_gather(t + NBUF + b, b)

        for b in range(NBUF):
            drain(sem_g, b)
            fire_out(N_CH - NBUF + b, b)
        for b in range(NBUF):
            drain(sem_o, b)

    return k(x_resh, table)


@jax.jit
def kernel(x, table):
    x_resh = x.reshape(NW, N_CH, CH)
    out = _sc_gather(x_resh, table)
    return out.reshape(BATCH, SEQ, EMBED)


# speedup vs baseline: 1.0135x; 1.0135x over previous
import functools

import jax
import jax.numpy as jnp
from jax import lax
from jax.experimental import pallas as pl
from jax.experimental.pallas import tpu as pltpu
from jax.experimental.pallas import tpu_sc as plsc

VOCAB = 100000
EMBED = 128
BATCH = 4096
SEQ = 200

NC = 2
NS = 16
NW = NC * NS

TOTAL = BATCH * SEQ
B_PER_W = TOTAL // NW
CH = 128
N_CH = B_PER_W // CH
NBUF = 6
GA = 3

MAIN_END = NBUF + ((N_CH - 2 * NBUF) // NBUF) * NBUF


def _sc_gather(x_resh, table):
    mesh = plsc.VectorSubcoreMesh(core_axis_name="c", subcore_axis_name="s")

    @functools.partial(
        pl.kernel,
        mesh=mesh,
        out_type=jax.ShapeDtypeStruct((NW, N_CH, CH, EMBED), jnp.float32),
        scratch_types=[
            pltpu.VMEM((N_CH, CH), jnp.int32),
            pltpu.VMEM((NBUF, CH, EMBED), jnp.float32),
            pltpu.SemaphoreType.DMA((NBUF,)),
            pltpu.SemaphoreType.DMA((NBUF,)),
        ],
    )
    def k(idx_hbm, table_hbm, out_hbm, idx_v, rows_v, sem_g, sem_o):
        wid = lax.axis_index("s") * NC + lax.axis_index("c")
        pltpu.sync_copy(idx_hbm.at[wid], idx_v)

        def fire_gather(chunk, b):
            pltpu.async_copy(table_hbm.at[idx_v.at[chunk]], rows_v.at[b],
                             sem_g.at[b])

        def fire_out(chunk, b):
            pltpu.async_copy(rows_v.at[b], out_hbm.at[wid, chunk],
                             sem_o.at[b])

        def drain(sem, b):
            pltpu.make_async_copy(out_hbm.at[wid, 0], rows_v.at[b],
                                  sem.at[b]).wait()

        def step_chunk(c, b, fire_ahead, drain_ahead):
            bf = (b + GA) % NBUF
            if drain_ahead:
                drain(sem_o, bf)
            if fire_ahead:
                fire_gather(c + GA, bf)
            drain(sem_g, b)
            fire_out(c, b)

        for c in range(GA):
            fire_gather(c, c)
        for c in range(NBUF):
            step_chunk(c, c, fire_ahead=True, drain_ahead=(c >= NBUF - GA))

        @pl.loop(NBUF, MAIN_END, step=NBUF)
        def _main(t):
            for b in range(NBUF):
                step_chunk(t + b, b, fire_ahead=True, drain_ahead=True)

        for c in range(MAIN_END, N_CH):
            step_chunk(c, c % NBUF, fire_ahead=(c + GA < N_CH),
                       drain_ahead=True)
        for c in range(N_CH - GA, N_CH):
            drain(sem_o, c % NBUF)

    return k(x_resh, table)


@jax.jit
def kernel(x, table):
    x_resh = x.reshape(NW, N_CH, CH)
    out = _sc_gather(x_resh, table)
    return out.reshape(BATCH, SEQ, EMBED)
